# baseline (speedup 1.0000x reference)
import functools

import jax
import jax.numpy as jnp
from jax import lax
from jax.experimental import pallas as pl
from jax.experimental.pallas import tpu as pltpu

_N_BINS = 15
_N_COLS = 1000
_BLOCK_C = 4096


def _tc_kernel(lt_ref, labels_ref, ece_ref, accs_ref, confs_ref,
               acc_scratch, *, n_rows, block_c):
    i = pl.program_id(0)
    g = pl.num_programs(0)

    x = lt_ref[...]
    m = jnp.max(x, axis=0, keepdims=True)
    s = jnp.sum(jnp.exp(x - m), axis=0, keepdims=True)
    conf = 1.0 / s
    lab = labels_ref[0, 0, :].reshape(1, block_c)
    rows = lax.broadcasted_iota(jnp.int32, x.shape, 0)
    xlab = jnp.max(jnp.where(rows == lab, x, -jnp.inf), axis=0, keepdims=True)
    acc = (xlab == m).astype(jnp.float32)

    idx = lax.broadcasted_iota(jnp.int32, (_N_BINS, 1), 0)
    idx_f = idx.astype(jnp.float32)
    step = jnp.float32(1.0) / jnp.float32(_N_BINS)
    lowers = idx_f * step
    uppers = jnp.where(idx == _N_BINS - 1, jnp.float32(1.0),
                       (idx_f + 1.0) * step)
    in_bin = ((conf > lowers) & (conf <= uppers)).astype(jnp.float32)
    counts = jnp.sum(in_bin, axis=1, keepdims=True)
    acc_sums = jnp.sum(acc * in_bin, axis=1, keepdims=True)
    conf_sums = jnp.sum(conf * in_bin, axis=1, keepdims=True)
    partial = jnp.concatenate([counts, acc_sums, conf_sums], axis=1)

    @pl.when(i == 0)
    def _init():
        acc_scratch[...] = partial

    @pl.when(i != 0)
    def _accum():
        acc_scratch[...] = acc_scratch[...] + partial

    @pl.when(i == g - 1)
    def _finalize():
        tot = acc_scratch[...]
        count = tot[:, 0:1]
        acc_sum = tot[:, 1:2]
        conf_sum = tot[:, 2:3]
        prop = count / float(n_rows)
        safe = jnp.maximum(count, 1.0)
        acc_bin = acc_sum / safe
        conf_bin = conf_sum / safe
        nonempty = count > 0.0
        gaps = jnp.where(nonempty, jnp.abs(conf_bin - acc_bin) * prop, 0.0)
        ece_ref[...] = jnp.sum(gaps, keepdims=True)
        accs_ref[...] = jnp.where(nonempty, acc_bin, jnp.nan)
        confs_ref[...] = jnp.where(nonempty, conf_bin, jnp.nan)


@jax.jit
def kernel(logits, labels):
    n_rows, n_cols = logits.shape
    lt = logits.T
    block_c = _BLOCK_C
    grid = n_rows // block_c
    labels3 = labels.reshape(grid, 1, block_c)

    ece, accs, confs = pl.pallas_call(
        functools.partial(_tc_kernel, n_rows=n_rows, block_c=block_c),
        grid=(grid,),
        in_specs=[
            pl.BlockSpec((n_cols, block_c), lambda i: (0, i)),
            pl.BlockSpec((1, 1, block_c), lambda i: (i, 0, 0)),
        ],
        out_specs=[
            pl.BlockSpec((1, 1), lambda i: (0, 0)),
            pl.BlockSpec((_N_BINS, 1), lambda i: (0, 0)),
            pl.BlockSpec((_N_BINS, 1), lambda i: (0, 0)),
        ],
        out_shape=[
            jax.ShapeDtypeStruct((1, 1), jnp.float32),
            jax.ShapeDtypeStruct((_N_BINS, 1), jnp.float32),
            jax.ShapeDtypeStruct((_N_BINS, 1), jnp.float32),
        ],
        scratch_shapes=[pltpu.VMEM((_N_BINS, 3), jnp.float32)],
    )(lt, labels3)
    return ece.reshape(1), accs.reshape(_N_BINS), confs.reshape(_N_BINS)


# speedup vs baseline: 1.0032x; 1.0032x over previous
import functools

import jax
import jax.numpy as jnp
from jax import lax
from jax.experimental import pallas as pl
from jax.experimental.pallas import tpu as pltpu

_N_BINS = 15
_N_COLS = 1000
_BLOCK_C = 1024


def _tc_kernel(lt_ref, labels_ref, ece_ref, accs_ref, confs_ref,
               acc_scratch, *, n_rows, block_c):
    i = pl.program_id(0)
    g = pl.num_programs(0)

    x = lt_ref[...]
    m = jnp.max(x, axis=0, keepdims=True)
    s = jnp.sum(jnp.exp(x - m), axis=0, keepdims=True)
    conf = 1.0 / s
    lab = labels_ref[0, 0, :].reshape(1, block_c)
    rows = lax.broadcasted_iota(jnp.int32, x.shape, 0)
    xlab = jnp.max(jnp.where(rows == lab, x, -jnp.inf), axis=0, keepdims=True)
    acc = (xlab == m).astype(jnp.float32)

    idx = lax.broadcasted_iota(jnp.int32, (_N_BINS, 1), 0)
    idx_f = idx.astype(jnp.float32)
    step = jnp.float32(1.0) / jnp.float32(_N_BINS)
    lowers = idx_f * step
    uppers = jnp.where(idx == _N_BINS - 1, jnp.float32(1.0),
                       (idx_f + 1.0) * step)
    in_bin = ((conf > lowers) & (conf <= uppers)).astype(jnp.float32)
    counts = jnp.sum(in_bin, axis=1, keepdims=True)
    acc_sums = jnp.sum(acc * in_bin, axis=1, keepdims=True)
    conf_sums = jnp.sum(conf * in_bin, axis=1, keepdims=True)
    partial = jnp.concatenate([counts, acc_sums, conf_sums], axis=1)

    @pl.when(i == 0)
    def _init():
        acc_scratch[...] = partial

    @pl.when(i != 0)
    def _accum():
        acc_scratch[...] = acc_scratch[...] + partial

    @pl.when(i == g - 1)
    def _finalize():
        tot = acc_scratch[...]
        count = tot[:, 0:1]
        acc_sum = tot[:, 1:2]
        conf_sum = tot[:, 2:3]
        prop = count / float(n_rows)
        safe = jnp.maximum(count, 1.0)
        acc_bin = acc_sum / safe
        conf_bin = conf_sum / safe
        nonempty = count > 0.0
        gaps = jnp.where(nonempty, jnp.abs(conf_bin - acc_bin) * prop, 0.0)
        ece_ref[...] = jnp.sum(gaps, keepdims=True)
        accs_ref[...] = jnp.where(nonempty, acc_bin, jnp.nan)
        confs_ref[...] = jnp.where(nonempty, conf_bin, jnp.nan)


@jax.jit
def kernel(logits, labels):
    n_rows, n_cols = logits.shape
    lt = logits.T
    block_c = _BLOCK_C
    grid = n_rows // block_c
    labels3 = labels.reshape(grid, 1, block_c)

    ece, accs, confs = pl.pallas_call(
        functools.partial(_tc_kernel, n_rows=n_rows, block_c=block_c),
        grid=(grid,),
        in_specs=[
            pl.BlockSpec((n_cols, block_c), lambda i: (0, i)),
            pl.BlockSpec((1, 1, block_c), lambda i: (i, 0, 0)),
        ],
        out_specs=[
            pl.BlockSpec((1, 1), lambda i: (0, 0)),
            pl.BlockSpec((_N_BINS, 1), lambda i: (0, 0)),
            pl.BlockSpec((_N_BINS, 1), lambda i: (0, 0)),
        ],
        out_shape=[
            jax.ShapeDtypeStruct((1, 1), jnp.float32),
            jax.ShapeDtypeStruct((_N_BINS, 1), jnp.float32),
            jax.ShapeDtypeStruct((_N_BINS, 1), jnp.float32),
        ],
        scratch_shapes=[pltpu.VMEM((_N_BINS, 3), jnp.float32)],
    )(lt, labels3)
    return ece.reshape(1), accs.reshape(_N_BINS), confs.reshape(_N_BINS)


# speedup vs baseline: 1.1063x; 1.1029x over previous
import functools

import jax
import jax.numpy as jnp
from jax import lax
from jax.experimental import pallas as pl
from jax.experimental.pallas import tpu as pltpu

_N_BINS = 15
_N_COLS = 1000
_BLOCK_C = 2048


def _tc_kernel(lt_ref, labels_ref, ece_ref, accs_ref, confs_ref,
               acc_scratch, *, n_rows, block_c):
    i = pl.program_id(0)
    g = pl.num_programs(0)

    x = lt_ref[...]
    m = jnp.max(x, axis=0, keepdims=True)
    s = jnp.sum(jnp.exp(x - m), axis=0, keepdims=True)
    conf = 1.0 / s
    lab = labels_ref[0, 0, :].reshape(1, block_c)
    rows = lax.broadcasted_iota(jnp.int32, x.shape, 0)
    xlab = jnp.max(jnp.where(rows == lab, x, -jnp.inf), axis=0, keepdims=True)
    acc = (xlab == m).astype(jnp.float32)

    idx = lax.broadcasted_iota(jnp.int32, (_N_BINS, 1), 0)
    idx_f = idx.astype(jnp.float32)
    step = jnp.float32(1.0) / jnp.float32(_N_BINS)
    lowers = idx_f * step
    uppers = jnp.where(idx == _N_BINS - 1, jnp.float32(1.0),
                       (idx_f + 1.0) * step)
    in_bin = ((conf > lowers) & (conf <= uppers)).astype(jnp.float32)
    counts = jnp.sum(in_bin, axis=1, keepdims=True)
    acc_sums = jnp.sum(acc * in_bin, axis=1, keepdims=True)
    conf_sums = jnp.sum(conf * in_bin, axis=1, keepdims=True)
    partial = jnp.concatenate([counts, acc_sums, conf_sums], axis=1)

    @pl.when(i == 0)
    def _init():
        acc_scratch[...] = partial

    @pl.when(i != 0)
    def _accum():
        acc_scratch[...] = acc_scratch[...] + partial

    @pl.when(i == g - 1)
    def _finalize():
        tot = acc_scratch[...]
        count = tot[:, 0:1]
        acc_sum = tot[:, 1:2]
        conf_sum = tot[:, 2:3]
        prop = count / float(n_rows)
        safe = jnp.maximum(count, 1.0)
        acc_bin = acc_sum / safe
        conf_bin = conf_sum / safe
        nonempty = count > 0.0
        gaps = jnp.where(nonempty, jnp.abs(conf_bin - acc_bin) * prop, 0.0)
        ece_ref[...] = jnp.sum(gaps, keepdims=True)
        accs_ref[...] = jnp.where(nonempty, acc_bin, jnp.nan)
        confs_ref[...] = jnp.where(nonempty, conf_bin, jnp.nan)


@jax.jit
def kernel(logits, labels):
    n_rows, n_cols = logits.shape
    lt = logits.T
    block_c = _BLOCK_C
    grid = n_rows // block_c
    labels3 = labels.reshape(grid, 1, block_c)

    ece, accs, confs = pl.pallas_call(
        functools.partial(_tc_kernel, n_rows=n_rows, block_c=block_c),
        grid=(grid,),
        in_specs=[
            pl.BlockSpec((n_cols, block_c), lambda i: (0, i)),
            pl.BlockSpec((1, 1, block_c), lambda i: (i, 0, 0)),
        ],
        out_specs=[
            pl.BlockSpec((1, 1), lambda i: (0, 0)),
            pl.BlockSpec((_N_BINS, 1), lambda i: (0, 0)),
            pl.BlockSpec((_N_BINS, 1), lambda i: (0, 0)),
        ],
        out_shape=[
            jax.ShapeDtypeStruct((1, 1), jnp.float32),
            jax.ShapeDtypeStruct((_N_BINS, 1), jnp.float32),
            jax.ShapeDtypeStruct((_N_BINS, 1), jnp.float32),
        ],
        scratch_shapes=[pltpu.VMEM((_N_BINS, 3), jnp.float32)],
    )(lt, labels3)
    return ece.reshape(1), accs.reshape(_N_BINS), confs.reshape(_N_BINS)
